# baseline (speedup 1.0000x reference)
import functools

import jax
import jax.numpy as jnp
from jax import lax
from jax.experimental import pallas as pl
from jax.experimental.pallas import tpu as pltpu
from jax.experimental.pallas import tpu_sc as plsc

N_NODE = 10000
E = 320000
H = 128
F_IN = 384

NC = 2
NS = 16
NW = NC * NS

EPT = E // NS
CH = 80
NCHUNK = EPT // CH
IBS = 50
NBLK = NCHUNK // IBS
EPT2 = 20480
CH_2 = 128
IBS2 = 20
NBLK2 = (EPT2 // CH_2) // IBS2
NPAD = 10240
ROWS_OUT = NPAD // NS
CROWS = NPAD // H
CNT_R = 128

EL = 100000
ELP = 100352
CH2 = 112
NCH2 = ELP // NW // CH2

_SC_MESH = dict(core_axis_name="c", subcore_axis_name="s",
                num_cores=NC, num_subcores=NS)


@functools.lru_cache(maxsize=None)
def _seg_sum_kernel(with_counts, ch, ibs, nblk, ept):
    mesh = plsc.VectorSubcoreMesh(**_SC_MESH)

    out_type = [
        jax.ShapeDtypeStruct((NPAD, H), jnp.float32),
        jax.ShapeDtypeStruct((NPAD, H), jnp.float32),
    ]
    scratch = [
        pltpu.VMEM((ibs * ch,), jnp.int32),
        pltpu.VMEM((ibs, ch), jnp.int32),
        pltpu.VMEM((ch, H), jnp.float32),
        pltpu.VMEM((ch, H), jnp.float32),
        pltpu.VMEM_SHARED((NPAD, H), jnp.float32),
        pltpu.SemaphoreType.DMA,
        pltpu.SemaphoreType.DMA,
        pltpu.SemaphoreType.DMA,
        pltpu.SemaphoreType.DMA,
    ]
    if with_counts:
        out_type = out_type + [
            jax.ShapeDtypeStruct((CNT_R, H), jnp.float32),
            jax.ShapeDtypeStruct((CNT_R, H), jnp.float32),
        ]
        scratch = scratch + [
            pltpu.VMEM((CROWS, H), jnp.float32),
            pltpu.VMEM((CROWS,), jnp.int32),
            pltpu.VMEM_SHARED((CNT_R, H), jnp.float32),
        ]

    @functools.partial(pl.kernel, out_type=tuple(out_type), mesh=mesh,
                       scratch_types=tuple(scratch),
                       compiler_params=pltpu.CompilerParams(needs_layout_passes=False))
    def seg(ems_hbm, emd_hbm, ets_hbm, etd_hbm, y_mt_hbm, y_tm_hbm, z_hbm, *rest):
        if with_counts:
            (out_mt, out_tm, cnt_t_hbm, cnt_m_hbm,
             sidx, didx, rows0, rows1, acc, sem0, sem1, sem2, sem3,
             hist, rowid, cnt_sh) = rest
        else:
            (out_mt, out_tm,
             sidx, didx, rows0, rows1, acc, sem0, sem1, sem2, sem3) = rest
        cid = lax.axis_index("c")
        sid = lax.axis_index("s")
        base = sid * ROWS_OUT

        pltpu.sync_copy(z_hbm, acc.at[pl.ds(base, ROWS_OUT)])
        if with_counts:
            pltpu.sync_copy(z_hbm.at[pl.ds(0, CROWS)], hist)
            pltpu.sync_copy(z_hbm.at[pl.ds(0, CNT_R // NS)],
                            cnt_sh.at[pl.ds(sid * (CNT_R // NS), CNT_R // NS)])
            lane16 = lax.broadcasted_iota(jnp.int32, (16,), 0)
            for r in range(CROWS // 16):
                rowid[pl.ds(r * 16, 16)] = lane16 + (r * 16)
        plsc.subcore_barrier()

        ones16 = jnp.full((16,), 1.0, jnp.float32)

        def run(es_hbm, ed_hbm, y_hbm, out_hbm, cnt_hbm):
            def count_chunk(j):
                for r in range(ch // 16):
                    idx = didx[j, pl.ds(r * 16, 16)]
                    plsc.addupdate_scatter(hist, [idx // H, idx % H], ones16)

            def gather(jl, buf, sem):
                return pltpu.make_async_copy(
                    y_hbm.at[sidx.at[pl.ds(jl * ch, ch)]], buf, sem)

            def scat(jl, buf, sem):
                return pltpu.async_copy(buf, acc.at[didx.at[jl]], sem,
                                        add=True)

            for b in range(nblk):
                pltpu.sync_copy(
                    es_hbm.at[pl.ds(sid * ept + b * (ibs * ch), ibs * ch)], sidx)
                pltpu.sync_copy(ed_hbm.at[sid, b], didx)
                gather(0, rows0, sem0).start()
                gather(1, rows1, sem1).start()

                def body(i, carry2):
                    j0 = 2 * i
                    j1 = j0 + 1
                    gather(j0, rows0, sem0).wait()
                    if with_counts:
                        count_chunk(j0)
                    s0 = scat(j0, rows0, sem2)
                    gather(j1, rows1, sem1).wait()
                    if with_counts:
                        count_chunk(j1)
                    s1 = scat(j1, rows1, sem3)
                    s0.wait()
                    gather(jnp.minimum(j0 + 2, ibs - 1), rows0, sem0).start()
                    s1.wait()
                    gather(jnp.minimum(j1 + 2, ibs - 1), rows1, sem1).start()
                    return carry2

                lax.fori_loop(0, ibs // 2, body, 0)
                gather(ibs - 1, rows0, sem0).wait()
                gather(ibs - 1, rows1, sem1).wait()

            if with_counts:
                pltpu.sync_copy(hist, cnt_sh.at[rowid], add=True)
            plsc.subcore_barrier()
            pltpu.sync_copy(acc.at[pl.ds(base, ROWS_OUT)],
                            out_hbm.at[pl.ds(base, ROWS_OUT)])
            if with_counts:
                nr = CNT_R // NS
                pltpu.sync_copy(cnt_sh.at[pl.ds(sid * nr, nr)],
                                cnt_hbm.at[pl.ds(sid * nr, nr)])

        @pl.when(cid == 0)
        def _():
            run(ems_hbm, emd_hbm, y_mt_hbm, out_mt,
                cnt_t_hbm if with_counts else None)

        @pl.when(cid == 1)
        def _():
            run(ets_hbm, etd_hbm, y_tm_hbm, out_tm,
                cnt_m_hbm if with_counts else None)

    return seg


@functools.lru_cache(maxsize=None)
def _label_dot_kernel():
    @functools.partial(
        pl.kernel,
        out_type=jax.ShapeDtypeStruct((NW, NCH2 * CH2), jnp.float32),
        mesh=plsc.VectorSubcoreMesh(**_SC_MESH),
        scratch_types=[
            pltpu.VMEM((NCH2, CH2), jnp.int32),
            pltpu.VMEM((NCH2, CH2), jnp.int32),
            pltpu.VMEM((CH2, H), jnp.float32),
            pltpu.VMEM((CH2, H), jnp.float32),
            pltpu.VMEM((CH2, H), jnp.float32),
            pltpu.VMEM((CH2, H), jnp.float32),
            pltpu.VMEM((16, 16), jnp.float32),
            pltpu.VMEM((NCH2 * CH2,), jnp.float32),
            pltpu.SemaphoreType.DMA,
            pltpu.SemaphoreType.DMA,
        ],
        compiler_params=pltpu.CompilerParams(needs_layout_passes=False),
    )
    def _label_dot(t2_hbm, m2_hbm, lt_hbm, lm_hbm, out_hbm,
                   idx_t, idx_m, tb0, mb0, tb1, mb1, tpb, ob, semA, semB):
        cid = lax.axis_index("c")
        sid = lax.axis_index("s")
        wid = cid * NS + sid
        pltpu.sync_copy(lt_hbm.at[wid], idx_t)
        pltpu.sync_copy(lm_hbm.at[wid], idx_m)
        lane = lax.broadcasted_iota(jnp.int32, (16,), 0)

        def fetch(c, tb, mb, sem):
            return (pltpu.make_async_copy(t2_hbm.at[idx_t.at[c]], tb, sem),
                    pltpu.make_async_copy(m2_hbm.at[idx_m.at[c]], mb, sem))

        def compute(c, tb, mb):
            def group(g, carry2):
                for e in range(16):
                    row = g * 16 + e
                    p = tb[row, pl.ds(0, 16)] * mb[row, pl.ds(0, 16)]
                    for k in range(1, H // 16):
                        p = p + tb[row, pl.ds(k * 16, 16)] * mb[row, pl.ds(k * 16, 16)]
                    plsc.store_scatter(tpb, [lane, jnp.full((16,), e, jnp.int32)], p)
                s = tpb[0, pl.ds(0, 16)]
                for k in range(1, 16):
                    s = s + tpb[k, pl.ds(0, 16)]
                ob[pl.ds(c * CH2 + g * 16, 16)] = s
                return carry2

            lax.fori_loop(0, CH2 // 16, group, 0)

        for d in fetch(0, tb0, mb0, semA):
            d.start()

        def body(i, carry):
            c0 = 2 * i
            c1 = c0 + 1
            for d in fetch(c0, tb0, mb0, semA):
                d.wait()
            for d in fetch(c1, tb1, mb1, semB):
                d.start()
            compute(c0, tb0, mb0)
            for d in fetch(c1, tb1, mb1, semB):
                d.wait()
            for d in fetch(jnp.minimum(c0 + 2, NCH2 - 1), tb0, mb0, semA):
                d.start()
            compute(c1, tb1, mb1)
            return carry

        lax.fori_loop(0, NCH2 // 2, body, 0)
        for d in fetch(NCH2 - 1, tb0, mb0, semA):
            d.wait()
        pltpu.sync_copy(ob, out_hbm.at[wid])

    return _label_dot


BT = 1000
GRID = N_NODE // BT


def _k1a_body(x_ref, embt_ref, embm_ref, wlin_ref, blin_ref,
              wl1mt_ref, wl1tm_ref,
              ymt1_ref, ytm1_ref, xt_ref):
    xt = (jnp.dot(x_ref[...], wlin_ref[...], preferred_element_type=jnp.float32)
          + blin_ref[...] + embt_ref[...])
    xm = embm_ref[...]
    ytm1_ref[...] = jnp.dot(xt, wl1tm_ref[...], preferred_element_type=jnp.float32)
    ymt1_ref[...] = jnp.dot(xm, wl1mt_ref[...], preferred_element_type=jnp.float32)
    xt_ref[...] = xt


def _k1b_body(xt_ref, embm_ref, wr1mt_ref, wr1tm_ref, rt1_ref, rm1_ref):
    rt1_ref[...] = jnp.dot(xt_ref[...], wr1mt_ref[...], preferred_element_type=jnp.float32)
    rm1_ref[...] = jnp.dot(embm_ref[...], wr1tm_ref[...], preferred_element_type=jnp.float32)


def _k4a_body(aggt_ref, aggm_ref, cntt_ref, cntm_ref, rt1_ref, rm1_ref,
              bl1mt_ref, bl1tm_ref, wl2mt_ref, wl2tm_ref,
              ymt2_ref, ytm2_ref, t1_ref, m1_ref):
    inv_t = 1.0 / jnp.maximum(cntt_ref[...], 1.0)
    inv_m = 1.0 / jnp.maximum(cntm_ref[...], 1.0)
    t1 = jax.nn.relu(aggt_ref[...] * inv_t + bl1mt_ref[...] + rt1_ref[...])
    m1 = jax.nn.relu(aggm_ref[...] * inv_m + bl1tm_ref[...] + rm1_ref[...])
    ymt2_ref[...] = jnp.dot(m1, wl2mt_ref[...], preferred_element_type=jnp.float32)
    ytm2_ref[...] = jnp.dot(t1, wl2tm_ref[...], preferred_element_type=jnp.float32)
    t1_ref[...] = t1
    m1_ref[...] = m1


def _k4b_body(t1_ref, m1_ref, wr2mt_ref, wr2tm_ref, rt2_ref, rm2_ref):
    rt2_ref[...] = jnp.dot(t1_ref[...], wr2mt_ref[...], preferred_element_type=jnp.float32)
    rm2_ref[...] = jnp.dot(m1_ref[...], wr2tm_ref[...], preferred_element_type=jnp.float32)


def _k6_body(aggt2_ref, aggm2_ref, cntt_ref, cntm_ref, rt2_ref, rm2_ref,
             bl2mt_ref, bl2tm_ref, t2_ref, m2_ref):
    inv_t = 1.0 / jnp.maximum(cntt_ref[...], 1.0)
    inv_m = 1.0 / jnp.maximum(cntm_ref[...], 1.0)
    t2_ref[...] = aggt2_ref[...] * inv_t + bl2mt_ref[...] + rt2_ref[...]
    m2_ref[...] = aggm2_ref[...] * inv_m + bl2tm_ref[...] + rm2_ref[...]


def _row_spec(width):
    return pl.BlockSpec((BT, width), lambda i: (i, 0))


def _full_spec(r, c):
    return pl.BlockSpec((r, c), lambda i: (0, 0))


_k1a_call = pl.pallas_call(
    _k1a_body,
    grid=(GRID,),
    in_specs=[
        _row_spec(F_IN), _row_spec(H), _row_spec(H),
        _full_spec(F_IN, H), _full_spec(1, H),
        _full_spec(H, H), _full_spec(H, H),
    ],
    out_specs=[_row_spec(H)] * 3,
    out_shape=[jax.ShapeDtypeStruct((N_NODE, H), jnp.float32)] * 3,
)

_k1b_call = pl.pallas_call(
    _k1b_body,
    grid=(GRID,),
    in_specs=[
        _row_spec(H), _row_spec(H),
        _full_spec(H, H), _full_spec(H, H),
    ],
    out_specs=[_row_spec(H)] * 2,
    out_shape=[jax.ShapeDtypeStruct((N_NODE, H), jnp.float32)] * 2,
)

_k4a_call = pl.pallas_call(
    _k4a_body,
    grid=(GRID,),
    in_specs=[
        _row_spec(H), _row_spec(H),
        pl.BlockSpec((BT, 1), lambda i: (i, 0)),
        pl.BlockSpec((BT, 1), lambda i: (i, 0)),
        _row_spec(H), _row_spec(H),
        _full_spec(1, H), _full_spec(1, H),
        _full_spec(H, H), _full_spec(H, H),
    ],
    out_specs=[_row_spec(H)] * 4,
    out_shape=[
        jax.ShapeDtypeStruct((NPAD, H), jnp.float32),
        jax.ShapeDtypeStruct((NPAD, H), jnp.float32),
        jax.ShapeDtypeStruct((N_NODE, H), jnp.float32),
        jax.ShapeDtypeStruct((N_NODE, H), jnp.float32),
    ],
)

_k4b_call = pl.pallas_call(
    _k4b_body,
    grid=(GRID,),
    in_specs=[
        _row_spec(H), _row_spec(H),
        _full_spec(H, H), _full_spec(H, H),
    ],
    out_specs=[_row_spec(H)] * 2,
    out_shape=[jax.ShapeDtypeStruct((N_NODE, H), jnp.float32)] * 2,
)

_k6_call = pl.pallas_call(
    _k6_body,
    grid=(GRID,),
    in_specs=[
        _row_spec(H), _row_spec(H),
        pl.BlockSpec((BT, 1), lambda i: (i, 0)),
        pl.BlockSpec((BT, 1), lambda i: (i, 0)),
        _row_spec(H), _row_spec(H),
        _full_spec(1, H), _full_spec(1, H),
    ],
    out_specs=[_row_spec(H)] * 2,
    out_shape=[jax.ShapeDtypeStruct((N_NODE, H), jnp.float32)] * 2,
)


def kernel(x_thesis, thesis_node_id, mentor_node_id, edge_index_tm, edge_index_mt,
           edge_label_index, W_lin, b_lin, emb_thesis, emb_mentor,
           Wl1_mt, bl1_mt, Wr1_mt, Wl1_tm, bl1_tm, Wr1_tm,
           Wl2_mt, bl2_mt, Wr2_mt, Wl2_tm, bl2_tm, Wr2_tm):
    ems = edge_index_mt[0]
    emd = edge_index_mt[1].reshape(NS, NBLK, IBS, CH)
    ets = edge_index_tm[0]
    etd = edge_index_tm[1].reshape(NS, NBLK, IBS, CH)

    zeros_h = jnp.zeros((ROWS_OUT, H), jnp.float32)
    lpad = jnp.zeros((2, ELP - EL), jnp.int32)
    lidx = jnp.concatenate([edge_label_index, lpad], axis=1)
    lt = lidx[0].reshape(NW, NCH2, CH2)
    lm = lidx[1].reshape(NW, NCH2, CH2)

    y_mt1, y_tm1, x_t = _k1a_call(
        x_thesis, emb_thesis, emb_mentor, W_lin, b_lin.reshape(1, H),
        Wl1_mt, Wl1_tm)
    r_t1, r_m1 = _k1b_call(x_t, emb_mentor, Wr1_mt, Wr1_tm)

    agg_t1, agg_m1, cnt_t, cnt_m = _seg_sum_kernel(True, CH, IBS, NBLK, EPT)(
        ems, emd, ets, etd, y_mt1, y_tm1, zeros_h)
    cnt_t2d = cnt_t[:CROWS].reshape(NPAD, 1)
    cnt_m2d = cnt_m[:CROWS].reshape(NPAD, 1)

    y_mt2, y_tm2, t1, m1 = _k4a_call(
        agg_t1, agg_m1, cnt_t2d, cnt_m2d, r_t1, r_m1,
        bl1_mt.reshape(1, H), bl1_tm.reshape(1, H),
        Wl2_mt, Wl2_tm)
    r_t2, r_m2 = _k4b_call(t1, m1, Wr2_mt, Wr2_tm)

    agg_t2, agg_m2 = _seg_sum_kernel(False, CH, IBS, NBLK, EPT)(
        ems, emd, ets, etd, y_mt2, y_tm2, zeros_h)

    t2, m2 = _k6_call(agg_t2, agg_m2, cnt_t2d, cnt_m2d, r_t2, r_m2,
                      bl2_mt.reshape(1, H), bl2_tm.reshape(1, H))

    out = _label_dot_kernel()(t2, m2, lt, lm)
    return out.reshape(ELP)[:EL]


# speedup vs baseline: 1.0004x; 1.0004x over previous
import functools

import jax
import jax.numpy as jnp
from jax import lax
from jax.experimental import pallas as pl
from jax.experimental.pallas import tpu as pltpu
from jax.experimental.pallas import tpu_sc as plsc

N_NODE = 10000
E = 320000
H = 128
F_IN = 384

NC = 2
NS = 16
NW = NC * NS

EPT = E // NS
CH = 80
NCHUNK = EPT // CH
IBS = 50
NBLK = NCHUNK // IBS
EPT2 = 20480
CH_2 = 128
IBS2 = 20
NBLK2 = (EPT2 // CH_2) // IBS2
NPAD = 10240
ROWS_OUT = NPAD // NS
CROWS = NPAD // H
CNT_R = 128

EL = 100000
ELP = 100352
CH2 = 112
NCH2 = ELP // NW // CH2

_SC_MESH = dict(core_axis_name="c", subcore_axis_name="s",
                num_cores=NC, num_subcores=NS)


@functools.lru_cache(maxsize=None)
def _seg_sum_kernel(with_counts, ch, ibs, nblk, ept):
    mesh = plsc.VectorSubcoreMesh(**_SC_MESH)

    out_type = [
        jax.ShapeDtypeStruct((NPAD, H), jnp.float32),
        jax.ShapeDtypeStruct((NPAD, H), jnp.float32),
    ]
    scratch = [
        pltpu.VMEM((ibs * ch,), jnp.int32),
        pltpu.VMEM((ibs, ch), jnp.int32),
        pltpu.VMEM((ch, H), jnp.float32),
        pltpu.VMEM((ch, H), jnp.float32),
        pltpu.VMEM_SHARED((NPAD, H), jnp.float32),
        pltpu.SemaphoreType.DMA,
        pltpu.SemaphoreType.DMA,
        pltpu.SemaphoreType.DMA,
        pltpu.SemaphoreType.DMA,
    ]
    if with_counts:
        out_type = out_type + [
            jax.ShapeDtypeStruct((CNT_R, H), jnp.float32),
            jax.ShapeDtypeStruct((CNT_R, H), jnp.float32),
        ]
        scratch = scratch + [
            pltpu.VMEM((CROWS, H), jnp.float32),
            pltpu.VMEM((CROWS,), jnp.int32),
            pltpu.VMEM_SHARED((CNT_R, H), jnp.float32),
        ]

    @functools.partial(pl.kernel, out_type=tuple(out_type), mesh=mesh,
                       scratch_types=tuple(scratch),
                       compiler_params=pltpu.CompilerParams(needs_layout_passes=False))
    def seg(ems_hbm, emd_hbm, ets_hbm, etd_hbm, y_mt_hbm, y_tm_hbm, z_hbm, *rest):
        if with_counts:
            (out_mt, out_tm, cnt_t_hbm, cnt_m_hbm,
             sidx, didx, rows0, rows1, acc, sem0, sem1, sem2, sem3,
             hist, rowid, cnt_sh) = rest
        else:
            (out_mt, out_tm,
             sidx, didx, rows0, rows1, acc, sem0, sem1, sem2, sem3) = rest
        cid = lax.axis_index("c")
        sid = lax.axis_index("s")
        base = sid * ROWS_OUT

        pltpu.sync_copy(z_hbm, acc.at[pl.ds(base, ROWS_OUT)])
        if with_counts:
            pltpu.sync_copy(z_hbm.at[pl.ds(0, CROWS)], hist)
            pltpu.sync_copy(z_hbm.at[pl.ds(0, CNT_R // NS)],
                            cnt_sh.at[pl.ds(sid * (CNT_R // NS), CNT_R // NS)])
            lane16 = lax.broadcasted_iota(jnp.int32, (16,), 0)
            for r in range(CROWS // 16):
                rowid[pl.ds(r * 16, 16)] = lane16 + (r * 16)
        plsc.subcore_barrier()

        ones16 = jnp.full((16,), 1.0, jnp.float32)

        def run(es_hbm, ed_hbm, y_hbm, out_hbm, cnt_hbm):
            def count_chunk(j):
                for r in range(ch // 16):
                    idx = didx[j, pl.ds(r * 16, 16)]
                    plsc.addupdate_scatter(hist, [idx // H, idx % H], ones16)

            def gather(jl, buf, sem):
                return pltpu.make_async_copy(
                    y_hbm.at[sidx.at[pl.ds(jl * ch, ch)]], buf, sem)

            def scat(jl, buf, sem):
                return pltpu.async_copy(buf, acc.at[didx.at[jl]], sem,
                                        add=True)

            for b in range(nblk):
                pltpu.sync_copy(
                    es_hbm.at[pl.ds(sid * ept + b * (ibs * ch), ibs * ch)], sidx)
                pltpu.sync_copy(ed_hbm.at[sid, b], didx)
                gather(0, rows0, sem0).start()
                gather(1, rows1, sem1).start()

                def body(i, carry2):
                    j0 = 2 * i
                    j1 = j0 + 1
                    gather(j0, rows0, sem0).wait()
                    if with_counts:
                        count_chunk(j0)
                    s0 = scat(j0, rows0, sem2)
                    gather(j1, rows1, sem1).wait()
                    if with_counts:
                        count_chunk(j1)
                    s1 = scat(j1, rows1, sem3)
                    s0.wait()
                    gather(jnp.minimum(j0 + 2, ibs - 1), rows0, sem0).start()
                    s1.wait()
                    gather(jnp.minimum(j1 + 2, ibs - 1), rows1, sem1).start()
                    return carry2

                lax.fori_loop(0, ibs // 2, body, 0)
                gather(ibs - 1, rows0, sem0).wait()
                gather(ibs - 1, rows1, sem1).wait()

            if with_counts:
                pltpu.sync_copy(hist, cnt_sh.at[rowid], add=True)
            plsc.subcore_barrier()
            pltpu.sync_copy(acc.at[pl.ds(base, ROWS_OUT)],
                            out_hbm.at[pl.ds(base, ROWS_OUT)])
            if with_counts:
                nr = CNT_R // NS
                pltpu.sync_copy(cnt_sh.at[pl.ds(sid * nr, nr)],
                                cnt_hbm.at[pl.ds(sid * nr, nr)])

        @pl.when(cid == 0)
        def _():
            run(ems_hbm, emd_hbm, y_mt_hbm, out_mt,
                cnt_t_hbm if with_counts else None)

        @pl.when(cid == 1)
        def _():
            run(ets_hbm, etd_hbm, y_tm_hbm, out_tm,
                cnt_m_hbm if with_counts else None)

    return seg


@functools.lru_cache(maxsize=None)
def _label_dot_kernel():
    @functools.partial(
        pl.kernel,
        out_type=jax.ShapeDtypeStruct((NW, NCH2 * CH2), jnp.float32),
        mesh=plsc.VectorSubcoreMesh(**_SC_MESH),
        scratch_types=[
            pltpu.VMEM((NCH2, CH2), jnp.int32),
            pltpu.VMEM((NCH2, CH2), jnp.int32),
            pltpu.VMEM((CH2, H), jnp.float32),
            pltpu.VMEM((CH2, H), jnp.float32),
            pltpu.VMEM((CH2, H), jnp.float32),
            pltpu.VMEM((CH2, H), jnp.float32),
            pltpu.VMEM((16, 16), jnp.float32),
            pltpu.VMEM((NCH2 * CH2,), jnp.float32),
            pltpu.SemaphoreType.DMA,
            pltpu.SemaphoreType.DMA,
        ],
        compiler_params=pltpu.CompilerParams(needs_layout_passes=False),
    )
    def _label_dot(t2_hbm, m2_hbm, lt_hbm, lm_hbm, out_hbm,
                   idx_t, idx_m, tb0, mb0, tb1, mb1, tpb, ob, semA, semB):
        cid = lax.axis_index("c")
        sid = lax.axis_index("s")
        wid = cid * NS + sid
        pltpu.sync_copy(lt_hbm.at[wid], idx_t)
        pltpu.sync_copy(lm_hbm.at[wid], idx_m)
        lane = lax.broadcasted_iota(jnp.int32, (16,), 0)

        def fetch(c, tb, mb, sem):
            return (pltpu.make_async_copy(t2_hbm.at[idx_t.at[c]], tb, sem),
                    pltpu.make_async_copy(m2_hbm.at[idx_m.at[c]], mb, sem))

        def compute(c, tb, mb):
            def group(g, carry2):
                for e in range(16):
                    row = g * 16 + e
                    p = tb[row, pl.ds(0, 16)] * mb[row, pl.ds(0, 16)]
                    for k in range(1, H // 16):
                        p = p + tb[row, pl.ds(k * 16, 16)] * mb[row, pl.ds(k * 16, 16)]
                    plsc.store_scatter(tpb, [lane, jnp.full((16,), e, jnp.int32)], p)
                s = tpb[0, pl.ds(0, 16)]
                for k in range(1, 16):
                    s = s + tpb[k, pl.ds(0, 16)]
                ob[pl.ds(c * CH2 + g * 16, 16)] = s
                return carry2

            lax.fori_loop(0, CH2 // 16, group, 0)

        for d in fetch(0, tb0, mb0, semA):
            d.start()

        def body(i, carry):
            c0 = 2 * i
            c1 = c0 + 1
            for d in fetch(c0, tb0, mb0, semA):
                d.wait()
            for d in fetch(c1, tb1, mb1, semB):
                d.start()
            compute(c0, tb0, mb0)
            for d in fetch(c1, tb1, mb1, semB):
                d.wait()
            for d in fetch(jnp.minimum(c0 + 2, NCH2 - 1), tb0, mb0, semA):
                d.start()
            compute(c1, tb1, mb1)
            return carry

        lax.fori_loop(0, NCH2 // 2, body, 0)
        for d in fetch(NCH2 - 1, tb0, mb0, semA):
            d.wait()
        pltpu.sync_copy(ob, out_hbm.at[wid])

    return _label_dot


BT = 1000
GRID = N_NODE // BT


def _k1_body(x_ref, embt_ref, embm_ref, wlin_ref, blin_ref,
             wl1mt_ref, wl1tm_ref, wr1mt_ref, wr1tm_ref,
             ymt1_ref, ytm1_ref, rt1_ref, rm1_ref):
    xt = (jnp.dot(x_ref[...], wlin_ref[...], preferred_element_type=jnp.float32)
          + blin_ref[...] + embt_ref[...])
    xm = embm_ref[...]
    ytm1_ref[...] = jnp.dot(xt, wl1tm_ref[...], preferred_element_type=jnp.float32)
    rt1_ref[...] = jnp.dot(xt, wr1mt_ref[...], preferred_element_type=jnp.float32)
    ymt1_ref[...] = jnp.dot(xm, wl1mt_ref[...], preferred_element_type=jnp.float32)
    rm1_ref[...] = jnp.dot(xm, wr1tm_ref[...], preferred_element_type=jnp.float32)


def _k4_body(aggt_ref, aggm_ref, cntt_ref, cntm_ref, rt1_ref, rm1_ref,
             bl1mt_ref, bl1tm_ref,
             wl2mt_ref, wl2tm_ref, wr2mt_ref, wr2tm_ref,
             ymt2_ref, ytm2_ref, rt2_ref, rm2_ref):
    inv_t = 1.0 / jnp.maximum(cntt_ref[...], 1.0)
    inv_m = 1.0 / jnp.maximum(cntm_ref[...], 1.0)
    t1 = jax.nn.relu(aggt_ref[...] * inv_t + bl1mt_ref[...] + rt1_ref[...])
    m1 = jax.nn.relu(aggm_ref[...] * inv_m + bl1tm_ref[...] + rm1_ref[...])
    ymt2_ref[...] = jnp.dot(m1, wl2mt_ref[...], preferred_element_type=jnp.float32)
    ytm2_ref[...] = jnp.dot(t1, wl2tm_ref[...], preferred_element_type=jnp.float32)
    rt2_ref[...] = jnp.dot(t1, wr2mt_ref[...], preferred_element_type=jnp.float32)
    rm2_ref[...] = jnp.dot(m1, wr2tm_ref[...], preferred_element_type=jnp.float32)


def _k6_body(aggt2_ref, aggm2_ref, cntt_ref, cntm_ref, rt2_ref, rm2_ref,
             bl2mt_ref, bl2tm_ref, t2_ref, m2_ref):
    inv_t = 1.0 / jnp.maximum(cntt_ref[...], 1.0)
    inv_m = 1.0 / jnp.maximum(cntm_ref[...], 1.0)
    t2_ref[...] = aggt2_ref[...] * inv_t + bl2mt_ref[...] + rt2_ref[...]
    m2_ref[...] = aggm2_ref[...] * inv_m + bl2tm_ref[...] + rm2_ref[...]


def _row_spec(width):
    return pl.BlockSpec((BT, width), lambda i: (i, 0))


def _full_spec(r, c):
    return pl.BlockSpec((r, c), lambda i: (0, 0))


_k1_call = pl.pallas_call(
    _k1_body,
    grid=(GRID,),
    in_specs=[
        _row_spec(F_IN), _row_spec(H), _row_spec(H),
        _full_spec(F_IN, H), _full_spec(1, H),
        _full_spec(H, H), _full_spec(H, H), _full_spec(H, H), _full_spec(H, H),
    ],
    out_specs=[_row_spec(H)] * 4,
    out_shape=[jax.ShapeDtypeStruct((N_NODE, H), jnp.float32)] * 4,
)

_k4_call = pl.pallas_call(
    _k4_body,
    grid=(GRID,),
    in_specs=[
        _row_spec(H), _row_spec(H),
        pl.BlockSpec((BT, 1), lambda i: (i, 0)),
        pl.BlockSpec((BT, 1), lambda i: (i, 0)),
        _row_spec(H), _row_spec(H),
        _full_spec(1, H), _full_spec(1, H),
        _full_spec(H, H), _full_spec(H, H), _full_spec(H, H), _full_spec(H, H),
    ],
    out_specs=[_row_spec(H)] * 4,
    out_shape=[
        jax.ShapeDtypeStruct((NPAD, H), jnp.float32),
        jax.ShapeDtypeStruct((NPAD, H), jnp.float32),
        jax.ShapeDtypeStruct((N_NODE, H), jnp.float32),
        jax.ShapeDtypeStruct((N_NODE, H), jnp.float32),
    ],
)

_k6_call = pl.pallas_call(
    _k6_body,
    grid=(GRID,),
    in_specs=[
        _row_spec(H), _row_spec(H),
        pl.BlockSpec((BT, 1), lambda i: (i, 0)),
        pl.BlockSpec((BT, 1), lambda i: (i, 0)),
        _row_spec(H), _row_spec(H),
        _full_spec(1, H), _full_spec(1, H),
    ],
    out_specs=[_row_spec(H)] * 2,
    out_shape=[jax.ShapeDtypeStruct((N_NODE, H), jnp.float32)] * 2,
)


def kernel(x_thesis, thesis_node_id, mentor_node_id, edge_index_tm, edge_index_mt,
           edge_label_index, W_lin, b_lin, emb_thesis, emb_mentor,
           Wl1_mt, bl1_mt, Wr1_mt, Wl1_tm, bl1_tm, Wr1_tm,
           Wl2_mt, bl2_mt, Wr2_mt, Wl2_tm, bl2_tm, Wr2_tm):
    ems = edge_index_mt[0]
    emd = edge_index_mt[1].reshape(NS, NBLK, IBS, CH)
    ets = edge_index_tm[0]
    etd = edge_index_tm[1].reshape(NS, NBLK, IBS, CH)

    zeros_h = jnp.zeros((ROWS_OUT, H), jnp.float32)
    lpad = jnp.zeros((2, ELP - EL), jnp.int32)
    lidx = jnp.concatenate([edge_label_index, lpad], axis=1)
    lt = lidx[0].reshape(NW, NCH2, CH2)
    lm = lidx[1].reshape(NW, NCH2, CH2)

    y_mt1, y_tm1, r_t1, r_m1 = _k1_call(
        x_thesis, emb_thesis, emb_mentor, W_lin, b_lin.reshape(1, H),
        Wl1_mt, Wl1_tm, Wr1_mt, Wr1_tm)

    agg_t1, agg_m1, cnt_t, cnt_m = _seg_sum_kernel(True, CH, IBS, NBLK, EPT)(
        ems, emd, ets, etd, y_mt1, y_tm1, zeros_h)
    cnt_t2d = cnt_t[:CROWS].reshape(NPAD, 1)
    cnt_m2d = cnt_m[:CROWS].reshape(NPAD, 1)

    y_mt2, y_tm2, r_t2, r_m2 = _k4_call(
        agg_t1, agg_m1, cnt_t2d, cnt_m2d, r_t1, r_m1,
        bl1_mt.reshape(1, H), bl1_tm.reshape(1, H),
        Wl2_mt, Wl2_tm, Wr2_mt, Wr2_tm)

    agg_t2, agg_m2 = _seg_sum_kernel(False, CH, IBS, NBLK, EPT)(
        ems, emd, ets, etd, y_mt2, y_tm2, zeros_h)

    t2, m2 = _k6_call(agg_t2, agg_m2, cnt_t2d, cnt_m2d, r_t2, r_m2,
                      bl2_mt.reshape(1, H), bl2_tm.reshape(1, H))

    out = _label_dot_kernel()(t2, m2, lt, lm)
    return out.reshape(ELP)[:EL]


# speedup vs baseline: 1.0110x; 1.0106x over previous
import functools

import jax
import jax.numpy as jnp
from jax import lax
from jax.experimental import pallas as pl
from jax.experimental.pallas import tpu as pltpu
from jax.experimental.pallas import tpu_sc as plsc

N_NODE = 10000
E = 320000
H = 128
F_IN = 384

NC = 2
NS = 16
NW = NC * NS

EPT = E // NS
CH = 80
NCHUNK = EPT // CH
IBS = 50
NBLK = NCHUNK // IBS
NPAD = 10240
ROWS_OUT = NPAD // NS
CROWS = NPAD // H
CNT_R = 128

EL = 100000
ELP = 100352
CH2 = 112
NCH2 = ELP // NW // CH2

_SC_MESH = dict(core_axis_name="c", subcore_axis_name="s",
                num_cores=NC, num_subcores=NS)


@functools.lru_cache(maxsize=None)
def _seg_sum_kernel(with_counts, ch, ibs, nblk, ept):
    mesh = plsc.VectorSubcoreMesh(**_SC_MESH)

    out_type = [
        jax.ShapeDtypeStruct((NPAD, H), jnp.float32),
        jax.ShapeDtypeStruct((NPAD, H), jnp.float32),
    ]
    scratch = [
        pltpu.VMEM((ibs * ch,), jnp.int32),
        pltpu.VMEM((ibs, ch), jnp.int32),
        pltpu.VMEM((ch, H), jnp.float32),
        pltpu.VMEM((ch, H), jnp.float32),
        pltpu.VMEM_SHARED((NPAD, H), jnp.float32),
        pltpu.SemaphoreType.DMA,
        pltpu.SemaphoreType.DMA,
        pltpu.SemaphoreType.DMA,
        pltpu.SemaphoreType.DMA,
    ]
    if with_counts:
        out_type = out_type + [
            jax.ShapeDtypeStruct((CNT_R, H), jnp.float32),
            jax.ShapeDtypeStruct((CNT_R, H), jnp.float32),
        ]
        scratch = scratch + [
            pltpu.VMEM((CROWS, H), jnp.float32),
            pltpu.VMEM((CROWS,), jnp.int32),
            pltpu.VMEM_SHARED((CNT_R, H), jnp.float32),
        ]

    @functools.partial(pl.kernel, out_type=tuple(out_type), mesh=mesh,
                       scratch_types=tuple(scratch),
                       compiler_params=pltpu.CompilerParams(needs_layout_passes=False))
    def seg(ems_hbm, emd_hbm, ets_hbm, etd_hbm, y_mt_hbm, y_tm_hbm, z_hbm, *rest):
        if with_counts:
            (out_mt, out_tm, cnt_t_hbm, cnt_m_hbm,
             sidx, didx, rows0, rows1, acc, sem0, sem1, sem2, sem3,
             hist, rowid, cnt_sh) = rest
        else:
            (out_mt, out_tm,
             sidx, didx, rows0, rows1, acc, sem0, sem1, sem2, sem3) = rest
        cid = lax.axis_index("c")
        sid = lax.axis_index("s")
        base = sid * ROWS_OUT

        pltpu.sync_copy(z_hbm, acc.at[pl.ds(base, ROWS_OUT)])
        if with_counts:
            pltpu.sync_copy(z_hbm.at[pl.ds(0, CROWS)], hist)
            pltpu.sync_copy(z_hbm.at[pl.ds(0, CNT_R // NS)],
                            cnt_sh.at[pl.ds(sid * (CNT_R // NS), CNT_R // NS)])
            lane16 = lax.broadcasted_iota(jnp.int32, (16,), 0)
            for r in range(CROWS // 16):
                rowid[pl.ds(r * 16, 16)] = lane16 + (r * 16)
        plsc.subcore_barrier()

        ones16 = jnp.full((16,), 1.0, jnp.float32)

        def run(es_hbm, ed_hbm, y_hbm, out_hbm, cnt_hbm):
            def count_chunk(j):
                for r in range(ch // 16):
                    idx = didx[j, pl.ds(r * 16, 16)]
                    plsc.addupdate_scatter(hist, [idx // H, idx % H], ones16)

            def gather(jl, buf, sem):
                return pltpu.make_async_copy(
                    y_hbm.at[sidx.at[pl.ds(jl * ch, ch)]], buf, sem)

            def scat(jl, buf, sem):
                return pltpu.async_copy(buf, acc.at[didx.at[jl]], sem,
                                        add=True)

            for b in range(nblk):
                pltpu.sync_copy(
                    es_hbm.at[pl.ds(sid * ept + b * (ibs * ch), ibs * ch)], sidx)
                pltpu.sync_copy(ed_hbm.at[sid, b], didx)
                gather(0, rows0, sem0).start()
                gather(1, rows1, sem1).start()

                def body(i, carry2):
                    j0 = 2 * i
                    j1 = j0 + 1
                    gather(j0, rows0, sem0).wait()
                    s0 = scat(j0, rows0, sem2)
                    if with_counts:
                        count_chunk(j0)
                    gather(j1, rows1, sem1).wait()
                    s1 = scat(j1, rows1, sem3)
                    if with_counts:
                        count_chunk(j1)
                    s0.wait()
                    gather(jnp.minimum(j0 + 2, ibs - 1), rows0, sem0).start()
                    s1.wait()
                    gather(jnp.minimum(j1 + 2, ibs - 1), rows1, sem1).start()
                    return carry2

                lax.fori_loop(0, ibs // 2, body, 0)
                gather(ibs - 1, rows0, sem0).wait()
                gather(ibs - 1, rows1, sem1).wait()

            if with_counts:
                pltpu.sync_copy(hist, cnt_sh.at[rowid], add=True)
            plsc.subcore_barrier()
            pltpu.sync_copy(acc.at[pl.ds(base, ROWS_OUT)],
                            out_hbm.at[pl.ds(base, ROWS_OUT)])
            if with_counts:
                nr = CNT_R // NS
                pltpu.sync_copy(cnt_sh.at[pl.ds(sid * nr, nr)],
                                cnt_hbm.at[pl.ds(sid * nr, nr)])

        @pl.when(cid == 0)
        def _():
            run(ems_hbm, emd_hbm, y_mt_hbm, out_mt,
                cnt_t_hbm if with_counts else None)

        @pl.when(cid == 1)
        def _():
            run(ets_hbm, etd_hbm, y_tm_hbm, out_tm,
                cnt_m_hbm if with_counts else None)

    return seg


@functools.lru_cache(maxsize=None)
def _label_dot_kernel():
    @functools.partial(
        pl.kernel,
        out_type=jax.ShapeDtypeStruct((NW, NCH2 * CH2), jnp.float32),
        mesh=plsc.VectorSubcoreMesh(**_SC_MESH),
        scratch_types=[
            pltpu.VMEM((NCH2, CH2), jnp.int32),
            pltpu.VMEM((NCH2, CH2), jnp.int32),
            pltpu.VMEM((CH2, H), jnp.float32),
            pltpu.VMEM((CH2, H), jnp.float32),
            pltpu.VMEM((CH2, H), jnp.float32),
            pltpu.VMEM((CH2, H), jnp.float32),
            pltpu.VMEM((16, 16), jnp.float32),
            pltpu.VMEM((NCH2 * CH2,), jnp.float32),
            pltpu.SemaphoreType.DMA,
            pltpu.SemaphoreType.DMA,
        ],
        compiler_params=pltpu.CompilerParams(needs_layout_passes=False),
    )
    def _label_dot(t2_hbm, m2_hbm, lt_hbm, lm_hbm, out_hbm,
                   idx_t, idx_m, tb0, mb0, tb1, mb1, tpb, ob, semA, semB):
        cid = lax.axis_index("c")
        sid = lax.axis_index("s")
        wid = cid * NS + sid
        pltpu.sync_copy(lt_hbm.at[wid], idx_t)
        pltpu.sync_copy(lm_hbm.at[wid], idx_m)
        lane = lax.broadcasted_iota(jnp.int32, (16,), 0)

        def fetch(c, tb, mb, sem):
            return (pltpu.make_async_copy(t2_hbm.at[idx_t.at[c]], tb, sem),
                    pltpu.make_async_copy(m2_hbm.at[idx_m.at[c]], mb, sem))

        def compute(c, tb, mb):
            def group(g, carry2):
                for e in range(16):
                    row = g * 16 + e
                    p = tb[row, pl.ds(0, 16)] * mb[row, pl.ds(0, 16)]
                    for k in range(1, H // 16):
                        p = p + tb[row, pl.ds(k * 16, 16)] * mb[row, pl.ds(k * 16, 16)]
                    plsc.store_scatter(tpb, [lane, jnp.full((16,), e, jnp.int32)], p)
                s = tpb[0, pl.ds(0, 16)]
                for k in range(1, 16):
                    s = s + tpb[k, pl.ds(0, 16)]
                ob[pl.ds(c * CH2 + g * 16, 16)] = s
                return carry2

            lax.fori_loop(0, CH2 // 16, group, 0)

        for d in fetch(0, tb0, mb0, semA):
            d.start()

        def body(i, carry):
            c0 = 2 * i
            c1 = c0 + 1
            for d in fetch(c0, tb0, mb0, semA):
                d.wait()
            for d in fetch(c1, tb1, mb1, semB):
                d.start()
            compute(c0, tb0, mb0)
            for d in fetch(c1, tb1, mb1, semB):
                d.wait()
            for d in fetch(jnp.minimum(c0 + 2, NCH2 - 1), tb0, mb0, semA):
                d.start()
            compute(c1, tb1, mb1)
            return carry

        lax.fori_loop(0, NCH2 // 2, body, 0)
        for d in fetch(NCH2 - 1, tb0, mb0, semA):
            d.wait()
        pltpu.sync_copy(ob, out_hbm.at[wid])

    return _label_dot


BT = 1000
GRID = N_NODE // BT


def _k1_body(x_ref, embt_ref, embm_ref, wlin_ref, blin_ref,
             wl1mt_ref, wl1tm_ref, wr1mt_ref, wr1tm_ref,
             ymt1_ref, ytm1_ref, rt1_ref, rm1_ref):
    xt = (jnp.dot(x_ref[...], wlin_ref[...], preferred_element_type=jnp.float32)
          + blin_ref[...] + embt_ref[...])
    xm = embm_ref[...]
    ytm1_ref[...] = jnp.dot(xt, wl1tm_ref[...], preferred_element_type=jnp.float32)
    rt1_ref[...] = jnp.dot(xt, wr1mt_ref[...], preferred_element_type=jnp.float32)
    ymt1_ref[...] = jnp.dot(xm, wl1mt_ref[...], preferred_element_type=jnp.float32)
    rm1_ref[...] = jnp.dot(xm, wr1tm_ref[...], preferred_element_type=jnp.float32)


def _k4_body(aggt_ref, aggm_ref, cntt_ref, cntm_ref, rt1_ref, rm1_ref,
             bl1mt_ref, bl1tm_ref,
             wl2mt_ref, wl2tm_ref, wr2mt_ref, wr2tm_ref,
             ymt2_ref, ytm2_ref, rt2_ref, rm2_ref):
    inv_t = 1.0 / jnp.maximum(cntt_ref[...], 1.0)
    inv_m = 1.0 / jnp.maximum(cntm_ref[...], 1.0)
    t1 = jax.nn.relu(aggt_ref[...] * inv_t + bl1mt_ref[...] + rt1_ref[...])
    m1 = jax.nn.relu(aggm_ref[...] * inv_m + bl1tm_ref[...] + rm1_ref[...])
    ymt2_ref[...] = jnp.dot(m1, wl2mt_ref[...], preferred_element_type=jnp.float32)
    ytm2_ref[...] = jnp.dot(t1, wl2tm_ref[...], preferred_element_type=jnp.float32)
    rt2_ref[...] = jnp.dot(t1, wr2mt_ref[...], preferred_element_type=jnp.float32)
    rm2_ref[...] = jnp.dot(m1, wr2tm_ref[...], preferred_element_type=jnp.float32)


def _k6_body(aggt2_ref, aggm2_ref, cntt_ref, cntm_ref, rt2_ref, rm2_ref,
             bl2mt_ref, bl2tm_ref, t2_ref, m2_ref):
    inv_t = 1.0 / jnp.maximum(cntt_ref[...], 1.0)
    inv_m = 1.0 / jnp.maximum(cntm_ref[...], 1.0)
    t2_ref[...] = aggt2_ref[...] * inv_t + bl2mt_ref[...] + rt2_ref[...]
    m2_ref[...] = aggm2_ref[...] * inv_m + bl2tm_ref[...] + rm2_ref[...]


def _row_spec(width):
    return pl.BlockSpec((BT, width), lambda i: (i, 0))


def _full_spec(r, c):
    return pl.BlockSpec((r, c), lambda i: (0, 0))


_k1_call = pl.pallas_call(
    _k1_body,
    grid=(GRID,),
    in_specs=[
        _row_spec(F_IN), _row_spec(H), _row_spec(H),
        _full_spec(F_IN, H), _full_spec(1, H),
        _full_spec(H, H), _full_spec(H, H), _full_spec(H, H), _full_spec(H, H),
    ],
    out_specs=[_row_spec(H)] * 4,
    out_shape=[jax.ShapeDtypeStruct((N_NODE, H), jnp.float32)] * 4,
)

_k4_call = pl.pallas_call(
    _k4_body,
    grid=(GRID,),
    in_specs=[
        _row_spec(H), _row_spec(H),
        pl.BlockSpec((BT, 1), lambda i: (i, 0)),
        pl.BlockSpec((BT, 1), lambda i: (i, 0)),
        _row_spec(H), _row_spec(H),
        _full_spec(1, H), _full_spec(1, H),
        _full_spec(H, H), _full_spec(H, H), _full_spec(H, H), _full_spec(H, H),
    ],
    out_specs=[_row_spec(H)] * 4,
    out_shape=[
        jax.ShapeDtypeStruct((NPAD, H), jnp.float32),
        jax.ShapeDtypeStruct((NPAD, H), jnp.float32),
        jax.ShapeDtypeStruct((N_NODE, H), jnp.float32),
        jax.ShapeDtypeStruct((N_NODE, H), jnp.float32),
    ],
)

_k6_call = pl.pallas_call(
    _k6_body,
    grid=(GRID,),
    in_specs=[
        _row_spec(H), _row_spec(H),
        pl.BlockSpec((BT, 1), lambda i: (i, 0)),
        pl.BlockSpec((BT, 1), lambda i: (i, 0)),
        _row_spec(H), _row_spec(H),
        _full_spec(1, H), _full_spec(1, H),
    ],
    out_specs=[_row_spec(H)] * 2,
    out_shape=[jax.ShapeDtypeStruct((N_NODE, H), jnp.float32)] * 2,
)


def kernel(x_thesis, thesis_node_id, mentor_node_id, edge_index_tm, edge_index_mt,
           edge_label_index, W_lin, b_lin, emb_thesis, emb_mentor,
           Wl1_mt, bl1_mt, Wr1_mt, Wl1_tm, bl1_tm, Wr1_tm,
           Wl2_mt, bl2_mt, Wr2_mt, Wl2_tm, bl2_tm, Wr2_tm):
    ems = edge_index_mt[0]
    emd = edge_index_mt[1].reshape(NS, NBLK, IBS, CH)
    ets = edge_index_tm[0]
    etd = edge_index_tm[1].reshape(NS, NBLK, IBS, CH)

    zeros_h = jnp.zeros((ROWS_OUT, H), jnp.float32)
    lpad = jnp.zeros((2, ELP - EL), jnp.int32)
    lidx = jnp.concatenate([edge_label_index, lpad], axis=1)
    lt = lidx[0].reshape(NW, NCH2, CH2)
    lm = lidx[1].reshape(NW, NCH2, CH2)

    y_mt1, y_tm1, r_t1, r_m1 = _k1_call(
        x_thesis, emb_thesis, emb_mentor, W_lin, b_lin.reshape(1, H),
        Wl1_mt, Wl1_tm, Wr1_mt, Wr1_tm)

    agg_t1, agg_m1, cnt_t, cnt_m = _seg_sum_kernel(True, CH, IBS, NBLK, EPT)(
        ems, emd, ets, etd, y_mt1, y_tm1, zeros_h)
    cnt_t2d = cnt_t[:CROWS].reshape(NPAD, 1)
    cnt_m2d = cnt_m[:CROWS].reshape(NPAD, 1)

    y_mt2, y_tm2, r_t2, r_m2 = _k4_call(
        agg_t1, agg_m1, cnt_t2d, cnt_m2d, r_t1, r_m1,
        bl1_mt.reshape(1, H), bl1_tm.reshape(1, H),
        Wl2_mt, Wl2_tm, Wr2_mt, Wr2_tm)

    agg_t2, agg_m2 = _seg_sum_kernel(False, CH, IBS, NBLK, EPT)(
        ems, emd, ets, etd, y_mt2, y_tm2, zeros_h)

    t2, m2 = _k6_call(agg_t2, agg_m2, cnt_t2d, cnt_m2d, r_t2, r_m2,
                      bl2_mt.reshape(1, H), bl2_tm.reshape(1, H))

    out = _label_dot_kernel()(t2, m2, lt, lm)
    return out.reshape(ELP)[:EL]
